# baseline (speedup 1.0000x reference)
import functools

import jax
import jax.numpy as jnp
from jax import lax
from jax.experimental import pallas as pl
from jax.experimental.pallas import tpu as pltpu
from jax.experimental.pallas import tpu_sc as plsc

_NUM_E = 1000000
_DIM = 64
_GROUP = 128


def _sc_gather(weights, idx2d, scales, groups_total, nw):
    groups_per_w = groups_total // nw
    b_flat = groups_total * _GROUP
    mesh = plsc.VectorSubcoreMesh(core_axis_name="c", subcore_axis_name="s")

    @functools.partial(
        pl.kernel,
        mesh=mesh,
        out_type=[
            jax.ShapeDtypeStruct((b_flat, _DIM // 4), jnp.int32),
            jax.ShapeDtypeStruct((groups_total, _GROUP), jnp.float32),
        ],
        scratch_types=[
            pltpu.VMEM((groups_per_w, _GROUP), jnp.int32),
            pltpu.VMEM((_GROUP, _DIM // 4), jnp.int32),
            pltpu.VMEM((_GROUP,), jnp.float32),
            pltpu.SemaphoreType.DMA,
            pltpu.SemaphoreType.DMA,
        ],
        compiler_params=pltpu.CompilerParams(use_tc_tiling_on_sc=False),
    )
    def k(w_hbm, idx_hbm, s_hbm, out_w, out_s, idx_v, rows_v, sc_v, sem_w, sem_s):
        wid = lax.axis_index("s") * 2 + lax.axis_index("c")
        g0 = wid * groups_per_w
        pltpu.sync_copy(idx_hbm.at[pl.ds(g0, groups_per_w)], idx_v)

        def body(g, carry):
            row_idx = idx_v.at[g]
            cp_w = pltpu.async_copy(w_hbm.at[row_idx], rows_v, sem_w)
            cp_s = pltpu.async_copy(s_hbm.at[row_idx], sc_v, sem_s)
            cp_w.wait()
            cp_s.wait()
            pltpu.sync_copy(rows_v, out_w.at[pl.ds((g0 + g) * _GROUP, _GROUP)])
            pltpu.sync_copy(sc_v, out_s.at[g0 + g])
            return carry

        lax.fori_loop(0, groups_per_w, body, 0)

    return k(weights, idx2d, scales)


def _tc_dequant(gw, gs, rows_per_blk=2048):
    b_flat = gw.shape[0]

    def body(w_ref, s_ref, o_ref):
        o_ref[...] = w_ref[...].astype(jnp.float32) * s_ref[...]

    return pl.pallas_call(
        body,
        grid=(b_flat // rows_per_blk,),
        in_specs=[
            pl.BlockSpec((rows_per_blk, _DIM), lambda i: (i, 0)),
            pl.BlockSpec((rows_per_blk, 1), lambda i: (i, 0)),
        ],
        out_specs=pl.BlockSpec((rows_per_blk, _DIM), lambda i: (i, 0)),
        out_shape=jax.ShapeDtypeStruct((b_flat, _DIM), jnp.float32),
    )(gw, gs)


def kernel(x, weights, scales):
    batch, hist = x.shape
    b_flat = batch * hist
    groups_total = b_flat // _GROUP
    idx2d = x.reshape(groups_total, _GROUP)
    w32 = jax.lax.bitcast_convert_type(
        weights.reshape(_NUM_E, _DIM // 4, 4), jnp.int32
    )
    gw, gs = _sc_gather(w32, idx2d, scales, groups_total, nw=32)
    gw8 = jax.lax.bitcast_convert_type(gw, jnp.int8).reshape(b_flat, _DIM)
    out = _tc_dequant(gw8, gs.reshape(b_flat, 1))
    return out.reshape(batch, hist, _DIM)
